# Initial kernel scaffold; baseline (speedup 1.0000x reference)
import jax
import jax.numpy as jnp
from jax.experimental import pallas as pl


def kernel(x, edge_index, batch, W_rel1, b_rel1, W_root1, W_rel2, b_rel2, W_root2, W_rel3, b_rel3, W_root3, gamma1, beta1, gamma2, beta2, gamma3, beta3, W_lin1, b_lin1, W_lin2, b_lin2):
    raise NotImplementedError("write your pallas kernel here")


# speedup vs baseline: 5.5848x; 5.5848x over previous
import functools

import jax
import jax.numpy as jnp
from jax import lax
from jax.experimental import pallas as pl
from jax.experimental.pallas import tpu as pltpu
from jax.experimental.pallas import tpu_sc as plsc

N_NODES = 10000
N_GRAPHS = 64
NEG_SLOPE = 0.01
BN_EPS = 1e-5

CHUNK = 128
CPW = 160
E_PAD = 16 * CPW * CHUNK
GROUP = 4
ROWS_PAD = 10240
ROWS_PER_SUB = ROWS_PAD // 16


def _make_sc_agg():
    mesh = plsc.VectorSubcoreMesh(core_axis_name="c", subcore_axis_name="s")

    @functools.partial(
        pl.kernel,
        mesh=mesh,
        compiler_params=pltpu.CompilerParams(use_tc_tiling_on_sc=False),
        out_type=jax.ShapeDtypeStruct((2, ROWS_PAD, 64), jnp.float32),
        scratch_types=[
            pltpu.VMEM((CPW, CHUNK), jnp.int32),
            pltpu.VMEM((CPW, CHUNK), jnp.int32),
            pltpu.VMEM((GROUP, CHUNK, 64), jnp.float32),
            pltpu.VMEM_SHARED((ROWS_PAD, 64), jnp.float32),
            pltpu.SemaphoreType.DMA,
        ],
    )
    def agg(table_a, table_b, srcw, dstw, out, src_v, dst_v, rows_v, acc, sem):
        c = lax.axis_index("c")
        s = lax.axis_index("s")

        def zrow(r, carry):
            for cc in range(4):
                rows_v[0, r, pl.ds(cc * 16, 16)] = jnp.zeros((16,), jnp.float32)
            return carry

        lax.fori_loop(0, CHUNK, zrow, 0)
        for k2 in range(ROWS_PER_SUB // CHUNK):
            pltpu.sync_copy(
                rows_v.at[0],
                acc.at[pl.ds(s * ROWS_PER_SUB + k2 * CHUNK, CHUNK)],
            )

        pltpu.sync_copy(srcw.at[s], src_v)
        pltpu.sync_copy(dstw.at[s], dst_v)
        plsc.subcore_barrier()

        def do_groups(table):
            def group_body(g, carry):
                base = g * GROUP
                handles = []
                for k in range(GROUP):
                    handles.append(
                        pltpu.async_copy(
                            table.at[src_v.at[base + k]], rows_v.at[k], sem
                        )
                    )
                for h in handles:
                    h.wait()
                for k in range(GROUP):
                    pltpu.sync_copy(
                        rows_v.at[k], acc.at[dst_v.at[base + k]], add=True
                    )
                return carry

            lax.fori_loop(0, CPW // GROUP, group_body, 0)

        @pl.when(c == 0)
        def _():
            do_groups(table_a)

        @pl.when(c == 1)
        def _():
            do_groups(table_b)

        plsc.subcore_barrier()

        for k2 in range(ROWS_PER_SUB // CHUNK):
            r0 = s * ROWS_PER_SUB + k2 * CHUNK
            pltpu.sync_copy(acc.at[pl.ds(r0, CHUNK)], rows_v.at[k2 % GROUP])
            pltpu.sync_copy(rows_v.at[k2 % GROUP], out.at[c, pl.ds(r0, CHUNK)])

    return agg


_SC_AGG_CACHE = []


def _sc_agg(table_a, table_b, srcp, dstp):
    if not _SC_AGG_CACHE:
        _SC_AGG_CACHE.append(_make_sc_agg())
    return _SC_AGG_CACHE[0](table_a, table_b, srcp, dstp)


R_BLK = 1000
N_STEPS = N_NODES // R_BLK


def _layer_dense(partials, h, wrel, brel, wroot):
    win = h.shape[1]
    wout = wrel.shape[1]
    npart = len(partials)

    def body(*refs):
        p_refs = refs[:npart]
        h_ref, wrel_ref, brel_ref, wroot_ref, y_ref, st_ref, acc = refs[npart:]
        i = pl.program_id(0)
        parts = [jnp.concatenate([pr[0], pr[1]], axis=-1) for pr in p_refs]
        aggb = parts[0] if npart == 1 else jnp.concatenate(parts, axis=-1)
        y = (
            jnp.dot(aggb, wrel_ref[...], preferred_element_type=jnp.float32)
            + jnp.dot(h_ref[...], wroot_ref[...], preferred_element_type=jnp.float32)
            + brel_ref[...]
        )
        y = jnp.where(y >= 0, y, NEG_SLOPE * y)
        y_ref[...] = y

        @pl.when(i == 0)
        def _():
            acc[...] = jnp.zeros_like(acc)

        acc[0:1, :] += jnp.sum(y, axis=0, keepdims=True)
        acc[1:2, :] += jnp.sum(y * y, axis=0, keepdims=True)

        @pl.when(i == N_STEPS - 1)
        def _():
            st_ref[...] = acc[...]

    in_specs = [
        pl.BlockSpec((2, R_BLK, 64), lambda i: (0, i, 0)) for _ in range(npart)
    ] + [
        pl.BlockSpec((R_BLK, win), lambda i: (i, 0)),
        pl.BlockSpec((win, wout), lambda i: (0, 0)),
        pl.BlockSpec((1, wout), lambda i: (0, 0)),
        pl.BlockSpec((win, wout), lambda i: (0, 0)),
    ]
    out_specs = [
        pl.BlockSpec((R_BLK, wout), lambda i: (i, 0)),
        pl.BlockSpec((8, wout), lambda i: (0, 0)),
    ]
    return pl.pallas_call(
        body,
        grid=(N_STEPS,),
        in_specs=in_specs,
        out_specs=out_specs,
        out_shape=[
            jax.ShapeDtypeStruct((N_NODES, wout), jnp.float32),
            jax.ShapeDtypeStruct((8, wout), jnp.float32),
        ],
        scratch_shapes=[pltpu.VMEM((8, wout), jnp.float32)],
    )(*partials, h, wrel, brel, wroot)


def _bn_pool(y, stats, gamma, beta, brow3, bcol3, with_cnt):
    wout = y.shape[1]
    nrep = wout // 128

    def body(y_ref, st_ref, g_ref, b_ref, brow_ref, bcol_ref, *refs):
        if with_cnt:
            hn_ref, s_ref, mx_ref, cnt_ref, s_acc, mx_acc, cnt_acc = refs
        else:
            hn_ref, s_ref, mx_ref, s_acc, mx_acc = refs
        i = pl.program_id(0)
        mu = st_ref[0:1, :] / float(N_NODES)
        var = st_ref[1:2, :] / float(N_NODES) - mu * mu
        rsig = lax.rsqrt(var + BN_EPS)
        hb = (y_ref[...] - mu) * (rsig * g_ref[...]) + b_ref[...]
        hn_ref[...] = hb

        brow = brow_ref[0]
        oht = (
            lax.broadcasted_iota(jnp.int32, (N_GRAPHS, R_BLK), 0) == brow
        ).astype(jnp.float32)

        @pl.when(i == 0)
        def _():
            s_acc[...] = jnp.zeros_like(s_acc)
            mx_acc[...] = jnp.full_like(mx_acc, -jnp.inf)
            if with_cnt:
                cnt_acc[...] = jnp.zeros_like(cnt_acc)

        s_acc[...] += jnp.dot(oht, hb, preferred_element_type=jnp.float32)
        if with_cnt:
            cnt_acc[...] += jnp.broadcast_to(
                jnp.sum(oht, axis=1, keepdims=True), (N_GRAPHS, 128)
            )

        bcol = bcol_ref[0]
        for g in range(N_GRAPHS):
            cmp = bcol == g
            if nrep > 1:
                cmp = jnp.concatenate([cmp] * nrep, axis=1)
            mg = jnp.max(
                jnp.where(cmp, hb, -jnp.inf), axis=0, keepdims=True
            )
            mx_acc[g : g + 1, :] = jnp.maximum(mx_acc[g : g + 1, :], mg)

        @pl.when(i == N_STEPS - 1)
        def _():
            s_ref[...] = s_acc[...]
            mx_ref[...] = mx_acc[...]
            if with_cnt:
                cnt_ref[...] = cnt_acc[...]

    in_specs = [
        pl.BlockSpec((R_BLK, wout), lambda i: (i, 0)),
        pl.BlockSpec((8, wout), lambda i: (0, 0)),
        pl.BlockSpec((1, wout), lambda i: (0, 0)),
        pl.BlockSpec((1, wout), lambda i: (0, 0)),
        pl.BlockSpec((1, 1, R_BLK), lambda i: (i, 0, 0)),
        pl.BlockSpec((1, R_BLK, 128), lambda i: (i, 0, 0)),
    ]
    out_specs = [
        pl.BlockSpec((R_BLK, wout), lambda i: (i, 0)),
        pl.BlockSpec((N_GRAPHS, wout), lambda i: (0, 0)),
        pl.BlockSpec((N_GRAPHS, wout), lambda i: (0, 0)),
    ]
    out_shape = [
        jax.ShapeDtypeStruct((N_NODES, wout), jnp.float32),
        jax.ShapeDtypeStruct((N_GRAPHS, wout), jnp.float32),
        jax.ShapeDtypeStruct((N_GRAPHS, wout), jnp.float32),
    ]
    scratch = [
        pltpu.VMEM((N_GRAPHS, wout), jnp.float32),
        pltpu.VMEM((N_GRAPHS, wout), jnp.float32),
    ]
    if with_cnt:
        out_specs.append(pl.BlockSpec((N_GRAPHS, 128), lambda i: (0, 0)))
        out_shape.append(jax.ShapeDtypeStruct((N_GRAPHS, 128), jnp.float32))
        scratch.append(pltpu.VMEM((N_GRAPHS, 128), jnp.float32))

    return pl.pallas_call(
        body,
        grid=(N_STEPS,),
        in_specs=in_specs,
        out_specs=out_specs,
        out_shape=out_shape,
        scratch_shapes=scratch,
    )(y, stats, gamma, beta, brow3, bcol3)


def _head(s1, mx1, s2, mx2, s3, mx3, cnt, w1, b1, w2p, b2p):
    def body(s1r, mx1r, s2r, mx2r, s3r, mx3r, cntr, w1r, b1r, w2r, b2r, outr):
        cnt_col = jnp.maximum(cntr[:, 0:1], 1.0)
        pieces = []
        for mxr, sr in ((mx1r, s1r), (mx2r, s2r), (mx3r, s3r)):
            sv = sr[...]
            pieces += [mxr[...], sv / cnt_col, sv]
        z = jnp.concatenate(pieces, axis=1)
        t = jnp.dot(z, w1r[...], preferred_element_type=jnp.float32) + b1r[...]
        u = jnp.dot(t, w2r[...], preferred_element_type=jnp.float32) + b2r[...]
        mask = lax.broadcasted_iota(jnp.int32, u.shape, 1) < 2
        m = jnp.max(jnp.where(mask, u, -jnp.inf), axis=1, keepdims=True)
        e = jnp.where(mask, jnp.exp(u - m), 0.0)
        lse = jnp.log(jnp.sum(e, axis=1, keepdims=True)) + m
        outr[...] = u - lse

    return pl.pallas_call(
        body,
        out_shape=jax.ShapeDtypeStruct((N_GRAPHS, 128), jnp.float32),
    )(s1, mx1, s2, mx2, s3, mx3, cnt, w1, b1, w2p, b2p)


def kernel(x, edge_index, batch,
           W_rel1, b_rel1, W_root1,
           W_rel2, b_rel2, W_root2,
           W_rel3, b_rel3, W_root3,
           gamma1, beta1, gamma2, beta2, gamma3, beta3,
           W_lin1, b_lin1, W_lin2, b_lin2):
    src = edge_index[0]
    dst = edge_index[1]
    npad = E_PAD - src.shape[0]
    pad_src = (jnp.arange(npad, dtype=jnp.int32) * 97) % N_NODES
    pad_dst = N_NODES + (jnp.arange(npad, dtype=jnp.int32) % (ROWS_PAD - N_NODES))
    srcp = jnp.concatenate([src, pad_src]).reshape(16, CPW, CHUNK)
    dstp = jnp.concatenate([dst, pad_dst]).reshape(16, CPW, CHUNK)

    brow3 = batch.reshape(N_STEPS, 1, R_BLK)
    bcol3 = jnp.broadcast_to(batch[:, None], (N_NODES, 128)).reshape(
        N_STEPS, R_BLK, 128
    )

    g1 = gamma1.reshape(1, -1); be1 = beta1.reshape(1, -1)
    g2 = gamma2.reshape(1, -1); be2 = beta2.reshape(1, -1)
    g3 = gamma3.reshape(1, -1); be3 = beta3.reshape(1, -1)
    br1 = b_rel1.reshape(1, -1); br2 = b_rel2.reshape(1, -1)
    br3 = b_rel3.reshape(1, -1)
    b1 = b_lin1.reshape(1, -1)
    w2p = jnp.pad(W_lin2, ((0, 0), (0, 128 - W_lin2.shape[1])))
    b2p = jnp.pad(b_lin2, (0, 128 - b_lin2.shape[0])).reshape(1, -1)

    p1 = _sc_agg(x[:, :64], x[:, 64:], srcp, dstp)
    y1, st1 = _layer_dense([p1], x, W_rel1, br1, W_root1)
    h1, s1, mx1, cnt = _bn_pool(y1, st1, g1, be1, brow3, bcol3, True)

    p2 = _sc_agg(h1[:, :64], h1[:, 64:], srcp, dstp)
    y2, st2 = _layer_dense([p2], h1, W_rel2, br2, W_root2)
    h2, s2, mx2 = _bn_pool(y2, st2, g2, be2, brow3, bcol3, False)

    p3a = _sc_agg(h2[:, 0:64], h2[:, 64:128], srcp, dstp)
    p3b = _sc_agg(h2[:, 128:192], h2[:, 192:256], srcp, dstp)
    y3, st3 = _layer_dense([p3a, p3b], h2, W_rel3, br3, W_root3)
    _, s3, mx3 = _bn_pool(y3, st3, g3, be3, brow3, bcol3, False)

    out = _head(s1, mx1, s2, mx2, s3, mx3, cnt, W_lin1, b1, w2p, b2p)
    return out[:, :2]
